# baseline (speedup 1.0000x reference)
import functools

import jax
import jax.numpy as jnp
from jax import lax
from jax.experimental import pallas as pl
from jax.experimental.pallas import tpu as pltpu
from jax.experimental.pallas import tpu_sc as plsc

B = 16384
D = 32
H = 128
O = 5
BB = 2048
R = 128 // D


@functools.cache
def _gather_fn():
    info = plsc.get_sparse_core_info()
    NC, NS = info.num_cores, info.num_subcores
    NW = NC * NS
    b_per_w = B // NW
    mesh = plsc.VectorSubcoreMesh(core_axis_name="c", subcore_axis_name="s")

    @functools.partial(
        pl.kernel,
        mesh=mesh,
        out_type=[
            jax.ShapeDtypeStruct((B, 128), jnp.float32),
            jax.ShapeDtypeStruct((B, 128), jnp.float32),
        ],
        scratch_types=[
            pltpu.VMEM((b_per_w,), jnp.int32),
            pltpu.VMEM((b_per_w, 128), jnp.float32),
            pltpu.SemaphoreType.DMA,
        ],
    )
    def gather_k(utab, mtab, uq, mq, ue_out, me_out, idx_v, rows_v, sem):
        wid = lax.axis_index("s") * NC + lax.axis_index("c")
        base = wid * b_per_w
        pltpu.sync_copy(uq.at[pl.ds(base, b_per_w)], idx_v)
        pltpu.async_copy(utab.at[idx_v], rows_v, sem).wait()
        pltpu.sync_copy(rows_v, ue_out.at[pl.ds(base, b_per_w)])
        pltpu.sync_copy(mq.at[pl.ds(base, b_per_w)], idx_v)
        pltpu.async_copy(mtab.at[idx_v], rows_v, sem).wait()
        pltpu.sync_copy(rows_v, me_out.at[pl.ds(base, b_per_w)])

    return gather_k


def _mlp_body(ue, me, ur, mr, w1u, w1m, b1, w2t, b2, woutt, bout, out):
    uef = ue[...]
    mef = me[...]
    uri = ur[...]
    mri = mr[...]
    zero = jnp.zeros((BB, D), jnp.float32)
    ues = zero
    mes = zero
    for s in range(R):
        ues = ues + jnp.where(uri == s, uef[:, D * s:D * (s + 1)], zero)
        mes = mes + jnp.where(mri == s, mef[:, D * s:D * (s + 1)], zero)
    h1 = jnp.maximum(
        jnp.dot(ues, w1u[...], preferred_element_type=jnp.float32)
        + jnp.dot(mes, w1m[...], preferred_element_type=jnp.float32)
        + b1[...], 0.0)
    h2 = jnp.maximum(
        jnp.dot(h1, w2t[...], preferred_element_type=jnp.float32) + b2[...], 0.0)
    out[...] = jnp.dot(h2, woutt[...], preferred_element_type=jnp.float32) + bout[...]


def kernel(User_ID, Movie_ID, Rating, user_table, movie_table,
           W1, b1, W2, b2, Wout, bout):
    ut128 = user_table.reshape(-1, 128)
    mt128 = movie_table.reshape(-1, 128)
    uq = User_ID >> 2
    mq = Movie_ID >> 2
    ue, me = _gather_fn()(ut128, mt128, uq, mq)

    ur = (User_ID & 3).reshape(B, 1)
    mr = (Movie_ID & 3).reshape(B, 1)
    w1u = W1[:, :D].T
    w1m = W1[:, D:].T
    w2t = W2.T
    woutt = Wout.T

    out = pl.pallas_call(
        _mlp_body,
        grid=(B // BB,),
        in_specs=[
            pl.BlockSpec((BB, 128), lambda i: (i, 0)),
            pl.BlockSpec((BB, 128), lambda i: (i, 0)),
            pl.BlockSpec((BB, 1), lambda i: (i, 0)),
            pl.BlockSpec((BB, 1), lambda i: (i, 0)),
            pl.BlockSpec((D, H), lambda i: (0, 0)),
            pl.BlockSpec((D, H), lambda i: (0, 0)),
            pl.BlockSpec((1, H), lambda i: (0, 0)),
            pl.BlockSpec((H, H), lambda i: (0, 0)),
            pl.BlockSpec((1, H), lambda i: (0, 0)),
            pl.BlockSpec((H, O), lambda i: (0, 0)),
            pl.BlockSpec((1, O), lambda i: (0, 0)),
        ],
        out_specs=pl.BlockSpec((BB, O), lambda i: (i, 0)),
        out_shape=jax.ShapeDtypeStruct((B, O), jnp.float32),
    )(ue, me, ur, mr, w1u, w1m, b1.reshape(1, H), w2t, b2.reshape(1, H),
      woutt, bout.reshape(1, O))
    return out


# speedup vs baseline: 1.6188x; 1.6188x over previous
import functools

import jax
import jax.numpy as jnp
from jax import lax
from jax.experimental import pallas as pl
from jax.experimental.pallas import tpu as pltpu
from jax.experimental.pallas import tpu_sc as plsc

B = 16384
D = 32
H = 128
O = 5
BB = 2048
CH = 128


@functools.cache
def _gather_fn():
    info = plsc.get_sparse_core_info()
    NC, NS = info.num_cores, info.num_subcores
    NW = NC * NS
    b_per_w = B // NW
    mesh = plsc.VectorSubcoreMesh(core_axis_name="c", subcore_axis_name="s")

    @functools.partial(
        pl.kernel,
        mesh=mesh,
        out_type=jax.ShapeDtypeStruct((B, 128), jnp.float32),
        scratch_types=[
            pltpu.VMEM((b_per_w,), jnp.int32),
            pltpu.VMEM((b_per_w,), jnp.int32),
            pltpu.VMEM((CH, D), jnp.float32),
            pltpu.VMEM((CH, D), jnp.float32),
            pltpu.VMEM((CH, 128), jnp.float32),
            pltpu.SemaphoreType.DMA,
        ],
    )
    def gather_k(utab, mtab, uid, mid, em_out,
                 uidx_v, midx_v, urows, mrows, rows, sem):
        wid = lax.axis_index("s") * NC + lax.axis_index("c")
        base = wid * b_per_w
        pltpu.sync_copy(uid.at[pl.ds(base, b_per_w)], uidx_v)
        pltpu.sync_copy(mid.at[pl.ds(base, b_per_w)], midx_v)

        for c in range(b_per_w // CH):
            off = c * CH

            def body(g, carry):
                uvec = uidx_v[pl.ds(off + g * 16, 16)]
                mvec = midx_v[pl.ds(off + g * 16, 16)]
                for k in range(16):
                    pltpu.async_copy(utab.at[pl.ds(uvec[k], 1)],
                                     urows.at[pl.ds(g * 16 + k, 1)], sem)
                    pltpu.async_copy(mtab.at[pl.ds(mvec[k], 1)],
                                     mrows.at[pl.ds(g * 16 + k, 1)], sem)
                return carry

            lax.fori_loop(0, CH // 16, body, 0)
            pltpu.make_async_copy(utab.at[pl.ds(0, CH)], urows, sem).wait()
            pltpu.make_async_copy(mtab.at[pl.ds(0, CH)], mrows, sem).wait()

            def place(i, carry):
                for col in range(0, D, 16):
                    rows[i, pl.ds(col, 16)] = urows[i, pl.ds(col, 16)]
                    rows[i, pl.ds(D + col, 16)] = mrows[i, pl.ds(col, 16)]
                return carry

            lax.fori_loop(0, CH, place, 0)
            pltpu.sync_copy(rows, em_out.at[pl.ds(base + off, CH)])

    return gather_k


def _mlp_body(em, w1t, b1, w2t, b2, woutt, bout, out):
    x = em[...][:, :2 * D]
    h1 = jnp.maximum(
        jnp.dot(x, w1t[...], preferred_element_type=jnp.float32) + b1[...], 0.0)
    h2 = jnp.maximum(
        jnp.dot(h1, w2t[...], preferred_element_type=jnp.float32) + b2[...], 0.0)
    out[...] = jnp.dot(h2, woutt[...], preferred_element_type=jnp.float32) + bout[...]


def kernel(User_ID, Movie_ID, Rating, user_table, movie_table,
           W1, b1, W2, b2, Wout, bout):
    em = _gather_fn()(user_table, movie_table, User_ID, Movie_ID)

    w1t = W1.T
    w2t = W2.T
    woutt = Wout.T

    out = pl.pallas_call(
        _mlp_body,
        grid=(B // BB,),
        in_specs=[
            pl.BlockSpec((BB, 128), lambda i: (i, 0)),
            pl.BlockSpec((2 * D, H), lambda i: (0, 0)),
            pl.BlockSpec((1, H), lambda i: (0, 0)),
            pl.BlockSpec((H, H), lambda i: (0, 0)),
            pl.BlockSpec((1, H), lambda i: (0, 0)),
            pl.BlockSpec((H, O), lambda i: (0, 0)),
            pl.BlockSpec((1, O), lambda i: (0, 0)),
        ],
        out_specs=pl.BlockSpec((BB, O), lambda i: (i, 0)),
        out_shape=jax.ShapeDtypeStruct((B, O), jnp.float32),
    )(em, w1t, b1.reshape(1, H), w2t, b2.reshape(1, H),
      woutt, bout.reshape(1, O))
    return out


# speedup vs baseline: 1.6437x; 1.0154x over previous
import functools

import jax
import jax.numpy as jnp
from jax import lax
from jax.experimental import pallas as pl
from jax.experimental.pallas import tpu as pltpu
from jax.experimental.pallas import tpu_sc as plsc

B = 16384
D = 32
H = 128
O = 5
BB = 2048
CH = 128


@functools.cache
def _gather_fn():
    info = plsc.get_sparse_core_info()
    NC, NS = info.num_cores, info.num_subcores
    NW = NC * NS
    b_per_w = B // NW
    mesh = plsc.VectorSubcoreMesh(core_axis_name="c", subcore_axis_name="s")

    @functools.partial(
        pl.kernel,
        mesh=mesh,
        out_type=jax.ShapeDtypeStruct((B, 128), jnp.float32),
        scratch_types=[
            pltpu.VMEM((b_per_w,), jnp.int32),
            pltpu.VMEM((b_per_w,), jnp.int32),
            pltpu.VMEM((CH, D), jnp.float32),
            pltpu.VMEM((CH, D), jnp.float32),
            pltpu.VMEM((CH, 128), jnp.float32),
            pltpu.SemaphoreType.DMA,
        ],
        compiler_params=pltpu.CompilerParams(use_tc_tiling_on_sc=True),
    )
    def gather_k(utab, mtab, uid, mid, em_out,
                 uidx_v, midx_v, urows, mrows, rows, sem):
        wid = lax.axis_index("s") * NC + lax.axis_index("c")
        base = wid * b_per_w
        pltpu.sync_copy(uid.at[pl.ds(base, b_per_w)], uidx_v)
        pltpu.sync_copy(mid.at[pl.ds(base, b_per_w)], midx_v)

        for c in range(b_per_w // CH):
            off = c * CH

            def body(g, carry):
                uvec = uidx_v[pl.ds(off + g * 16, 16)]
                mvec = midx_v[pl.ds(off + g * 16, 16)]
                for k in range(16):
                    pltpu.async_copy(utab.at[pl.ds(uvec[k], 1)],
                                     urows.at[pl.ds(g * 16 + k, 1)], sem)
                    pltpu.async_copy(mtab.at[pl.ds(mvec[k], 1)],
                                     mrows.at[pl.ds(g * 16 + k, 1)], sem)
                return carry

            lax.fori_loop(0, CH // 16, body, 0)
            pltpu.make_async_copy(utab.at[pl.ds(0, CH)], urows, sem).wait()
            pltpu.make_async_copy(mtab.at[pl.ds(0, CH)], mrows, sem).wait()

            def place(i, carry):
                for col in range(0, D, 16):
                    rows[i, pl.ds(col, 16)] = urows[i, pl.ds(col, 16)]
                    rows[i, pl.ds(D + col, 16)] = mrows[i, pl.ds(col, 16)]
                return carry

            lax.fori_loop(0, CH, place, 0)
            pltpu.sync_copy(rows, em_out.at[pl.ds(base + off, CH)])

    return gather_k


def _mlp_body(em, w1t, b1, w2t, b2, woutt, bout, out):
    x = em[...][:, :2 * D]
    h1 = jnp.maximum(
        jnp.dot(x, w1t[...], preferred_element_type=jnp.float32) + b1[...], 0.0)
    h2 = jnp.maximum(
        jnp.dot(h1, w2t[...], preferred_element_type=jnp.float32) + b2[...], 0.0)
    out[...] = jnp.dot(h2, woutt[...], preferred_element_type=jnp.float32) + bout[...]


def kernel(User_ID, Movie_ID, Rating, user_table, movie_table,
           W1, b1, W2, b2, Wout, bout):
    em = _gather_fn()(user_table, movie_table, User_ID, Movie_ID)

    w1t = W1.T
    w2t = W2.T
    woutt = Wout.T

    out = pl.pallas_call(
        _mlp_body,
        grid=(B // BB,),
        in_specs=[
            pl.BlockSpec((BB, 128), lambda i: (i, 0)),
            pl.BlockSpec((2 * D, H), lambda i: (0, 0)),
            pl.BlockSpec((1, H), lambda i: (0, 0)),
            pl.BlockSpec((H, H), lambda i: (0, 0)),
            pl.BlockSpec((1, H), lambda i: (0, 0)),
            pl.BlockSpec((H, O), lambda i: (0, 0)),
            pl.BlockSpec((1, O), lambda i: (0, 0)),
        ],
        out_specs=pl.BlockSpec((BB, O), lambda i: (i, 0)),
        out_shape=jax.ShapeDtypeStruct((B, O), jnp.float32),
    )(em, w1t, b1.reshape(1, H), w2t, b2.reshape(1, H),
      woutt, bout.reshape(1, O))
    return out
